# baseline (speedup 1.0000x reference)
import functools

import jax
import jax.numpy as jnp
from jax.experimental import pallas as pl

ALPHA = 4
NUM_FRAMES = 32
BATCH = 4
SLOW_FRAMES = NUM_FRAMES // ALPHA
SLOW_IDX = tuple(
    int(i * (NUM_FRAMES - 1) / (SLOW_FRAMES - 1)) for i in range(SLOW_FRAMES)
)
_SLOT_OF_T = []
for _t in range(NUM_FRAMES):
    _k = 0
    for _j, _s in enumerate(SLOW_IDX):
        if _t >= _s:
            _k = _j
    _SLOT_OF_T.append(_k)

FRAME_ROWS = 3 * 224 * 224 // 128
LANES = 128


def _body(in_ref, fast_ref, slow_ref):
    r = pl.program_id(0)
    t = jax.lax.rem(r, NUM_FRAMES)
    data = in_ref[...]
    fast_ref[...] = data
    is_slow = functools.reduce(jnp.logical_or, [t == s for s in SLOW_IDX])

    @pl.when(is_slow)
    def _():
        slow_ref[...] = data


def kernel(frames):
    b, n, c, h, w = frames.shape
    flat = frames.reshape(b * n, FRAME_ROWS, LANES)

    def slow_index_map(r):
        bb = jax.lax.div(r, NUM_FRAMES)
        t = jax.lax.rem(r, NUM_FRAMES)
        slot = sum((t >= s).astype(jnp.int32) for s in SLOW_IDX[1:])
        return bb * SLOW_FRAMES + slot, 0, 0

    fast_flat, slow_flat = pl.pallas_call(
        _body,
        grid=(b * n,),
        in_specs=[pl.BlockSpec((1, FRAME_ROWS, LANES), lambda r: (r, 0, 0))],
        out_specs=[
            pl.BlockSpec((1, FRAME_ROWS, LANES), lambda r: (r, 0, 0)),
            pl.BlockSpec((1, FRAME_ROWS, LANES), slow_index_map),
        ],
        out_shape=[
            jax.ShapeDtypeStruct((b * n, FRAME_ROWS, LANES), frames.dtype),
            jax.ShapeDtypeStruct((b * SLOW_FRAMES, FRAME_ROWS, LANES), frames.dtype),
        ],
    )(flat)
    fast = fast_flat.reshape(b, n, c, h, w)
    slow = slow_flat.reshape(b, SLOW_FRAMES, c, h, w)
    return (slow, fast)
